# baseline (speedup 1.0000x reference)
import functools

import jax
import jax.numpy as jnp
from jax import lax
from jax.experimental import pallas as pl
from jax.experimental.pallas import tpu as pltpu
from jax.experimental.pallas import tpu_sc as plsc

N = 10000
E = 320000
D_IN = 128
D_HID = 128
D_LAT = 64

NT = N + 8
NCORE = 2
NSUB = 16
NW = NCORE * NSUB
RPT = 626
N_ACC = NSUB * RPT
EP = NW * 10240
TPB = EP // NW

_f32 = jnp.float32


def _make_agg(D, Bx, NB, split=None):
    iters = TPB // Bx
    it0 = iters if split is None else split
    it1 = 2 * iters - it0
    assert TPB % Bx == 0 and it0 % (2 * NB) == 0 and it1 % (2 * NB) == 0, (D, Bx, NB)
    pertile = NB * (Bx * D + 4 * Bx) + 64
    assert 16 * pertile + N_ACC * D < 2**21 - 16384, (D, Bx, NB)
    mesh = plsc.VectorSubcoreMesh(core_axis_name="c", subcore_axis_name="s")
    out_type = jax.ShapeDtypeStruct((NCORE, N_ACC, D), _f32)
    scratch = (
        [pltpu.VMEM((2, Bx), jnp.int32) for _ in range(2 * NB)]
        + [pltpu.VMEM((Bx, D), _f32) for _ in range(NB)]
        + [pltpu.VMEM_SHARED((N_ACC, D), _f32)]
        + [pltpu.SemaphoreType.DMA for _ in range(4 * NB)]
    )

    def body(table, eidx, zeros2, out, *rest):
        idxA = rest[:NB]
        idxB = rest[NB:2 * NB]
        rows = rest[2 * NB:3 * NB]
        acc = rest[3 * NB]
        sems = rest[3 * NB + 1:]
        isemA = sems[:NB]
        isemB = sems[NB:2 * NB]
        gsem = sems[2 * NB:3 * NB]
        ssem = sems[3 * NB:4 * NB]
        c = lax.axis_index("c")
        s = lax.axis_index("s")
        w = c * NSUB + s
        rbase = s * RPT
        g0 = jnp.where(c == 0, s * it0, NSUB * it0 + s * it1)
        nstep = jnp.where(c == 0, it0 // (2 * NB), it1 // (2 * NB))
        glast = NSUB * (it0 + it1) - 1

        pltpu.sync_copy(zeros2.at[pl.ds(rbase, RPT)], acc.at[pl.ds(rbase, RPT)])
        for b in range(NB):
            pltpu.async_copy(eidx.at[g0 + b], idxA[b], isemA[b])
        plsc.subcore_barrier()

        def half(gbase, idx, isem, prefetch_g, pidx, psem):
            for b in range(NB):
                pltpu.make_async_copy(eidx.at[gbase + b], idx[b], isem[b]).wait()
            for b in range(NB):
                pltpu.async_copy(eidx.at[lax.min(prefetch_g + b, glast)],
                                 pidx[b], psem[b])
            for b in range(NB):
                pltpu.async_copy(table.at[idx[b].at[0]], rows[b], gsem[b])
            for b in range(NB):
                pltpu.make_async_copy(table.at[idx[b].at[0]], rows[b],
                                      gsem[b]).wait()
                pltpu.async_copy(rows[b], acc.at[idx[b].at[1]], ssem[b],
                                 add=True)
            for b in range(NB):
                pltpu.make_async_copy(rows[b], acc.at[idx[b].at[1]],
                                      ssem[b]).wait()

        def step(j, carry):
            ga = g0 + 2 * j * NB
            gb = ga + NB
            half(ga, idxA, isemA, gb, idxB, isemB)
            half(gb, idxB, isemB, ga + 2 * NB, idxA, isemA)
            return carry

        lax.fori_loop(0, nstep, step, 0)
        for b in range(NB):
            pltpu.make_async_copy(eidx.at[g0], idxA[b], isemA[b]).wait()
        plsc.subcore_barrier()
        pltpu.sync_copy(acc.at[pl.ds(rbase, RPT)], out.at[c, pl.ds(rbase, RPT)])

    return pl.kernel(body, out_type=out_type, mesh=mesh, scratch_types=scratch,
                     compiler_params=pltpu.CompilerParams(use_tc_tiling_on_sc=False),
                     name=f"sc_agg_d{D}")


D_AUG = 144
_agg144 = _make_agg(D_AUG, 64, 4, split=80)
_agg128 = _make_agg(128, 80, 4, split=64)
_agg64 = _make_agg(64, 128, 4, split=40)


def _eidx(srcp, dstp, Bx):
    return jnp.stack([srcp.reshape(-1, Bx), dstp.reshape(-1, Bx)], axis=1)


def _k1_body(agg, x, w1l, b1, w1r, w2l, h_ref, m2_ref, inv_ref):
    cnt = agg[0, :N, 128] + agg[1, :N, 128]
    inv = 1.0 / jnp.maximum(cnt, 1.0)
    inv_ref[...] = inv
    a = (agg[0, :N, :128] + agg[1, :N, :128]) * inv[:, None]
    h = a @ w1l[...] + b1[...][None, :] + x[...] @ w1r[...]
    h = jnp.maximum(h, 0.0)
    h_ref[:N, :] = h
    h_ref[N:, :] = jnp.zeros((NT - N, D_HID), _f32)
    m2_ref[:N, :] = h @ w2l[...]
    m2_ref[N:, :] = jnp.zeros((NT - N, D_LAT), _f32)


def _k2_body(s2, inv, h, w2r, b2, z_ref):
    z = ((s2[0, :N, :] + s2[1, :N, :]) * inv[...][:, None]
         + b2[...][None, :] + h[:N, :] @ w2r[...])
    z_ref[:N, :] = z
    z_ref[N:, :] = jnp.zeros((NT - N, D_LAT), _f32)


def _k3_body(agg3, inv, z, w3l, b3, w3r, w4l, h2_ref, m4_ref):
    a = (agg3[0, :N, :] + agg3[1, :N, :]) * inv[...][:, None]
    h2 = a @ w3l[...] + b3[...][None, :] + z[:N, :] @ w3r[...]
    h2 = jnp.maximum(h2, 0.0)
    h2_ref[:N, :] = h2
    h2_ref[N:, :] = jnp.zeros((NT - N, D_HID), _f32)
    m4_ref[:N, :] = h2 @ w4l[...]
    m4_ref[N:, :] = jnp.zeros((NT - N, D_IN), _f32)


def _k4_body(s4, inv, h2, w4r, b4, out_ref):
    out_ref[...] = ((s4[0, :N, :] + s4[1, :N, :]) * inv[...][:, None]
                    + b4[...][None, :] + h2[:N, :] @ w4r[...])


_k1 = pl.pallas_call(
    _k1_body,
    out_shape=[jax.ShapeDtypeStruct((NT, D_HID), _f32),
               jax.ShapeDtypeStruct((NT, D_LAT), _f32),
               jax.ShapeDtypeStruct((N,), _f32)])
_k2 = pl.pallas_call(
    _k2_body,
    out_shape=jax.ShapeDtypeStruct((NT, D_LAT), _f32))
_k3 = pl.pallas_call(
    _k3_body,
    out_shape=[jax.ShapeDtypeStruct((NT, D_HID), _f32),
               jax.ShapeDtypeStruct((NT, D_IN), _f32)])
_k4 = pl.pallas_call(
    _k4_body,
    out_shape=jax.ShapeDtypeStruct((N, D_IN), _f32))


def kernel(x, edge_index, W1l, b1, W1r, W2l, b2, W2r, W3l, b3, W3r, W4l, b4, W4r):
    src = edge_index[0]
    dst = edge_index[1]
    pad = jnp.full((EP - E,), N, jnp.int32)
    srcp = jnp.concatenate([src, pad])
    dstp = jnp.concatenate([dst, pad])
    xt = jnp.concatenate([x, jnp.zeros((NT - N, D_IN), _f32)])
    zeros2_144 = jnp.zeros((N_ACC, D_AUG), _f32)
    zeros2_128 = jnp.zeros((N_ACC, 128), _f32)
    zeros2_64 = jnp.zeros((N_ACC, 64), _f32)
    xa = jnp.concatenate(
        [xt, jnp.concatenate([jnp.ones((N, 1), _f32), jnp.zeros((NT - N, 1), _f32)]),
         jnp.zeros((NT, D_AUG - 129), _f32)], axis=1)

    e64 = _eidx(srcp, dstp, 64)
    e80 = _eidx(srcp, dstp, 80)
    e128 = _eidx(srcp, dstp, 128)

    agg1 = _agg144(xa, e64, zeros2_144)
    h, m2, inv = _k1(agg1, x, W1l, b1, W1r, W2l)
    s2 = _agg64(m2, e128, zeros2_64)
    z = _k2(s2, inv, h, W2r, b2)
    agg3 = _agg64(z, e128, zeros2_64)
    h2, m4 = _k3(agg3, inv, z, W3l, b3, W3r, W4l)
    s4 = _agg128(m4, e80, zeros2_128)
    x_hat = _k4(s4, inv, h2, W4r, b4)
    return x_hat


# speedup vs baseline: 1.2251x; 1.2251x over previous
import functools

import jax
import jax.numpy as jnp
from jax import lax
from jax.experimental import pallas as pl
from jax.experimental.pallas import tpu as pltpu
from jax.experimental.pallas import tpu_sc as plsc

N = 10000
E = 320000
D_IN = 128
D_HID = 128
D_LAT = 64

NT = N + 8
NCORE = 2
NSUB = 16
NW = NCORE * NSUB
RPT = 626
N_ACC = NSUB * RPT
EP = NW * 10240
TPB = EP // NW

_f32 = jnp.float32


def _make_agg(D, Bx, NB, split=None):
    iters = TPB // Bx
    it0 = iters if split is None else split
    it1 = 2 * iters - it0
    assert TPB % Bx == 0 and it0 % (2 * NB) == 0 and it1 % (2 * NB) == 0, (D, Bx, NB)
    pertile = NB * (Bx * D + 4 * Bx) + 64
    assert 16 * pertile + N_ACC * D < 2**21 - 16384, (D, Bx, NB)
    mesh = plsc.VectorSubcoreMesh(core_axis_name="c", subcore_axis_name="s")
    out_type = jax.ShapeDtypeStruct((NCORE, N_ACC, D), _f32)
    scratch = (
        [pltpu.VMEM((2, Bx), jnp.int32) for _ in range(2 * NB)]
        + [pltpu.VMEM((Bx, D), _f32) for _ in range(NB)]
        + [pltpu.VMEM_SHARED((N_ACC, D), _f32)]
        + [pltpu.SemaphoreType.DMA for _ in range(4 * NB)]
    )

    def body(table, eidx, zeros2, out, *rest):
        idxA = rest[:NB]
        idxB = rest[NB:2 * NB]
        rows = rest[2 * NB:3 * NB]
        acc = rest[3 * NB]
        sems = rest[3 * NB + 1:]
        isemA = sems[:NB]
        isemB = sems[NB:2 * NB]
        gsem = sems[2 * NB:3 * NB]
        ssem = sems[3 * NB:4 * NB]
        c = lax.axis_index("c")
        s = lax.axis_index("s")
        w = c * NSUB + s
        rbase = s * RPT
        g0 = jnp.where(c == 0, s * it0, NSUB * it0 + s * it1)
        nstep = jnp.where(c == 0, it0 // (2 * NB), it1 // (2 * NB))
        glast = NSUB * (it0 + it1) - 1

        pltpu.sync_copy(zeros2.at[pl.ds(rbase, RPT)], acc.at[pl.ds(rbase, RPT)])
        for b in range(NB):
            pltpu.async_copy(eidx.at[g0 + b], idxA[b], isemA[b])
        plsc.subcore_barrier()

        def half(gbase, idx, isem, prefetch_g, pidx, psem):
            for b in range(NB):
                pltpu.make_async_copy(eidx.at[gbase + b], idx[b], isem[b]).wait()
            for b in range(NB):
                pltpu.async_copy(eidx.at[lax.min(prefetch_g + b, glast)],
                                 pidx[b], psem[b])
            for b in range(NB):
                pltpu.async_copy(table.at[idx[b].at[0]], rows[b], gsem[b])
            for b in range(NB):
                pltpu.make_async_copy(table.at[idx[b].at[0]], rows[b],
                                      gsem[b]).wait()
                pltpu.async_copy(rows[b], acc.at[idx[b].at[1]], ssem[b],
                                 add=True)
            for b in range(NB):
                pltpu.make_async_copy(rows[b], acc.at[idx[b].at[1]],
                                      ssem[b]).wait()

        def step(j, carry):
            ga = g0 + 2 * j * NB
            gb = ga + NB
            half(ga, idxA, isemA, gb, idxB, isemB)
            half(gb, idxB, isemB, ga + 2 * NB, idxA, isemA)
            return carry

        lax.fori_loop(0, nstep, step, 0)
        for b in range(NB):
            pltpu.make_async_copy(eidx.at[g0], idxA[b], isemA[b]).wait()
        plsc.subcore_barrier()
        pltpu.sync_copy(acc.at[pl.ds(rbase, RPT)], out.at[c, pl.ds(rbase, RPT)])

    return pl.kernel(body, out_type=out_type, mesh=mesh, scratch_types=scratch,
                     compiler_params=pltpu.CompilerParams(use_tc_tiling_on_sc=False),
                     name=f"sc_agg_d{D}")


D_AUG = 144
_agg144 = _make_agg(D_AUG, 64, 4, split=240)
_agg128 = _make_agg(128, 80, 4, split=192)
_agg64 = _make_agg(64, 128, 4, split=120)


def _eidx(srcp, dstp, Bx):
    return jnp.stack([srcp.reshape(-1, Bx), dstp.reshape(-1, Bx)], axis=1)


def _k1_body(agg, x, w1l, b1, w1r, w2l, h_ref, m2_ref, inv_ref):
    cnt = agg[0, :N, 128] + agg[1, :N, 128]
    inv = 1.0 / jnp.maximum(cnt, 1.0)
    inv_ref[...] = inv
    a = (agg[0, :N, :128] + agg[1, :N, :128]) * inv[:, None]
    h = a @ w1l[...] + b1[...][None, :] + x[...] @ w1r[...]
    h = jnp.maximum(h, 0.0)
    h_ref[:N, :] = h
    h_ref[N:, :] = jnp.zeros((NT - N, D_HID), _f32)
    m2_ref[:N, :] = h @ w2l[...]
    m2_ref[N:, :] = jnp.zeros((NT - N, D_LAT), _f32)


def _k2_body(s2, inv, h, w2r, b2, z_ref):
    z = ((s2[0, :N, :] + s2[1, :N, :]) * inv[...][:, None]
         + b2[...][None, :] + h[:N, :] @ w2r[...])
    z_ref[:N, :] = z
    z_ref[N:, :] = jnp.zeros((NT - N, D_LAT), _f32)


def _k3_body(agg3, inv, z, w3l, b3, w3r, w4l, h2_ref, m4_ref):
    a = (agg3[0, :N, :] + agg3[1, :N, :]) * inv[...][:, None]
    h2 = a @ w3l[...] + b3[...][None, :] + z[:N, :] @ w3r[...]
    h2 = jnp.maximum(h2, 0.0)
    h2_ref[:N, :] = h2
    h2_ref[N:, :] = jnp.zeros((NT - N, D_HID), _f32)
    m4_ref[:N, :] = h2 @ w4l[...]
    m4_ref[N:, :] = jnp.zeros((NT - N, D_IN), _f32)


def _k4_body(s4, inv, h2, w4r, b4, out_ref):
    out_ref[...] = ((s4[0, :N, :] + s4[1, :N, :]) * inv[...][:, None]
                    + b4[...][None, :] + h2[:N, :] @ w4r[...])


_k1 = pl.pallas_call(
    _k1_body,
    out_shape=[jax.ShapeDtypeStruct((NT, D_HID), _f32),
               jax.ShapeDtypeStruct((NT, D_LAT), _f32),
               jax.ShapeDtypeStruct((N,), _f32)])
_k2 = pl.pallas_call(
    _k2_body,
    out_shape=jax.ShapeDtypeStruct((NT, D_LAT), _f32))
_k3 = pl.pallas_call(
    _k3_body,
    out_shape=[jax.ShapeDtypeStruct((NT, D_HID), _f32),
               jax.ShapeDtypeStruct((NT, D_IN), _f32)])
_k4 = pl.pallas_call(
    _k4_body,
    out_shape=jax.ShapeDtypeStruct((N, D_IN), _f32))


def kernel(x, edge_index, W1l, b1, W1r, W2l, b2, W2r, W3l, b3, W3r, W4l, b4, W4r):
    src = edge_index[0]
    dst = edge_index[1]
    pad = jnp.full((EP - E,), N, jnp.int32)
    srcp = jnp.concatenate([src, pad])
    dstp = jnp.concatenate([dst, pad])
    xt = jnp.concatenate([x, jnp.zeros((NT - N, D_IN), _f32)])
    zeros2_144 = jnp.zeros((N_ACC, D_AUG), _f32)
    zeros2_128 = jnp.zeros((N_ACC, 128), _f32)
    zeros2_64 = jnp.zeros((N_ACC, 64), _f32)
    xa = jnp.concatenate(
        [xt, jnp.concatenate([jnp.ones((N, 1), _f32), jnp.zeros((NT - N, 1), _f32)]),
         jnp.zeros((NT, D_AUG - 129), _f32)], axis=1)

    e64 = _eidx(srcp, dstp, 64)
    e80 = _eidx(srcp, dstp, 80)
    e128 = _eidx(srcp, dstp, 128)

    agg1 = _agg144(xa, e64, zeros2_144)
    h, m2, inv = _k1(agg1, x, W1l, b1, W1r, W2l)
    s2 = _agg64(m2, e128, zeros2_64)
    z = _k2(s2, inv, h, W2r, b2)
    agg3 = _agg64(z, e128, zeros2_64)
    h2, m4 = _k3(agg3, inv, z, W3l, b3, W3r, W4l)
    s4 = _agg128(m4, e80, zeros2_128)
    x_hat = _k4(s4, inv, h2, W4r, b4)
    return x_hat


# speedup vs baseline: 1.2661x; 1.0335x over previous
import functools

import jax
import jax.numpy as jnp
from jax import lax
from jax.experimental import pallas as pl
from jax.experimental.pallas import tpu as pltpu
from jax.experimental.pallas import tpu_sc as plsc

N = 10000
E = 320000
D_IN = 128
D_HID = 128
D_LAT = 64

NT = N + 8
NCORE = 2
NSUB = 16
NW = NCORE * NSUB
RPT = 626
N_ACC = NSUB * RPT
EP = NW * 10240
TPB = EP // NW

_f32 = jnp.float32


def _make_agg(D, Bx, NB, split=None):
    iters = TPB // Bx
    it0 = iters if split is None else split
    it1 = 2 * iters - it0
    assert TPB % Bx == 0 and it0 % (2 * NB) == 0 and it1 % (2 * NB) == 0, (D, Bx, NB)
    pertile = NB * (Bx * D + 4 * Bx) + 64
    assert 16 * pertile + N_ACC * D < 2**21 - 16384, (D, Bx, NB)
    mesh = plsc.VectorSubcoreMesh(core_axis_name="c", subcore_axis_name="s")
    out_type = jax.ShapeDtypeStruct((NCORE, N_ACC, D), _f32)
    scratch = (
        [pltpu.VMEM((2, Bx), jnp.int32) for _ in range(2 * NB)]
        + [pltpu.VMEM((Bx, D), _f32) for _ in range(NB)]
        + [pltpu.VMEM_SHARED((N_ACC, D), _f32)]
        + [pltpu.SemaphoreType.DMA for _ in range(4 * NB)]
    )

    def body(table, eidx, zeros2, out, *rest):
        idxA = rest[:NB]
        idxB = rest[NB:2 * NB]
        rows = rest[2 * NB:3 * NB]
        acc = rest[3 * NB]
        sems = rest[3 * NB + 1:]
        isemA = sems[:NB]
        isemB = sems[NB:2 * NB]
        gsem = sems[2 * NB:3 * NB]
        ssem = sems[3 * NB:4 * NB]
        c = lax.axis_index("c")
        s = lax.axis_index("s")
        w = c * NSUB + s
        rbase = s * RPT
        g0 = jnp.where(c == 0, s * it0, NSUB * it0 + s * it1)
        nstep = jnp.where(c == 0, it0 // (2 * NB), it1 // (2 * NB))
        glast = NSUB * (it0 + it1) - 1

        pltpu.sync_copy(zeros2.at[pl.ds(rbase, RPT)], acc.at[pl.ds(rbase, RPT)])
        for b in range(NB):
            pltpu.async_copy(eidx.at[g0 + b], idxA[b], isemA[b])
        plsc.subcore_barrier()

        def half(gbase, idx, isem, prefetch_g, pidx, psem):
            for b in range(NB):
                pltpu.make_async_copy(eidx.at[gbase + b], idx[b], isem[b]).wait()
            for b in range(NB):
                pltpu.async_copy(eidx.at[lax.min(prefetch_g + b, glast)],
                                 pidx[b], psem[b])
            for b in range(NB):
                pltpu.async_copy(table.at[idx[b].at[0]], rows[b], gsem[b])
            for b in range(NB):
                pltpu.make_async_copy(table.at[idx[b].at[0]], rows[b],
                                      gsem[b]).wait()
                pltpu.async_copy(rows[b], acc.at[idx[b].at[1]], ssem[b],
                                 add=True)
            for b in range(NB):
                pltpu.make_async_copy(rows[b], acc.at[idx[b].at[1]],
                                      ssem[b]).wait()

        def step(j, carry):
            ga = g0 + 2 * j * NB
            gb = ga + NB
            half(ga, idxA, isemA, gb, idxB, isemB)
            half(gb, idxB, isemB, ga + 2 * NB, idxA, isemA)
            return carry

        lax.fori_loop(0, nstep, step, 0)
        for b in range(NB):
            pltpu.make_async_copy(eidx.at[g0], idxA[b], isemA[b]).wait()
        plsc.subcore_barrier()
        pltpu.sync_copy(acc.at[pl.ds(rbase, RPT)], out.at[c, pl.ds(rbase, RPT)])

    return pl.kernel(body, out_type=out_type, mesh=mesh, scratch_types=scratch,
                     compiler_params=pltpu.CompilerParams(use_tc_tiling_on_sc=False),
                     name=f"sc_agg_d{D}")


D_AUG = 144
_agg144 = _make_agg(D_AUG, 64, 4, split=256)
_agg128 = _make_agg(128, 80, 4, split=200)
_agg64 = _make_agg(64, 128, 4, split=128)


def _eidx(srcp, dstp, Bx):
    return jnp.stack([srcp.reshape(-1, Bx), dstp.reshape(-1, Bx)], axis=1)


def _k1_body(agg, x, w1l, b1, w1r, w2l, h_ref, m2_ref, inv_ref):
    cnt = agg[0, :N, 128] + agg[1, :N, 128]
    inv = 1.0 / jnp.maximum(cnt, 1.0)
    inv_ref[...] = inv
    a = (agg[0, :N, :128] + agg[1, :N, :128]) * inv[:, None]
    h = a @ w1l[...] + b1[...][None, :] + x[...] @ w1r[...]
    h = jnp.maximum(h, 0.0)
    h_ref[:N, :] = h
    h_ref[N:, :] = jnp.zeros((NT - N, D_HID), _f32)
    m2_ref[:N, :] = h @ w2l[...]
    m2_ref[N:, :] = jnp.zeros((NT - N, D_LAT), _f32)


def _k2_body(s2, inv, h, w2r, b2, z_ref):
    z = ((s2[0, :N, :] + s2[1, :N, :]) * inv[...][:, None]
         + b2[...][None, :] + h[:N, :] @ w2r[...])
    z_ref[:N, :] = z
    z_ref[N:, :] = jnp.zeros((NT - N, D_LAT), _f32)


def _k3_body(agg3, inv, z, w3l, b3, w3r, w4l, h2_ref, m4_ref):
    a = (agg3[0, :N, :] + agg3[1, :N, :]) * inv[...][:, None]
    h2 = a @ w3l[...] + b3[...][None, :] + z[:N, :] @ w3r[...]
    h2 = jnp.maximum(h2, 0.0)
    h2_ref[:N, :] = h2
    h2_ref[N:, :] = jnp.zeros((NT - N, D_HID), _f32)
    m4_ref[:N, :] = h2 @ w4l[...]
    m4_ref[N:, :] = jnp.zeros((NT - N, D_IN), _f32)


def _k4_body(s4, inv, h2, w4r, b4, out_ref):
    out_ref[...] = ((s4[0, :N, :] + s4[1, :N, :]) * inv[...][:, None]
                    + b4[...][None, :] + h2[:N, :] @ w4r[...])


_k1 = pl.pallas_call(
    _k1_body,
    out_shape=[jax.ShapeDtypeStruct((NT, D_HID), _f32),
               jax.ShapeDtypeStruct((NT, D_LAT), _f32),
               jax.ShapeDtypeStruct((N,), _f32)])
_k2 = pl.pallas_call(
    _k2_body,
    out_shape=jax.ShapeDtypeStruct((NT, D_LAT), _f32))
_k3 = pl.pallas_call(
    _k3_body,
    out_shape=[jax.ShapeDtypeStruct((NT, D_HID), _f32),
               jax.ShapeDtypeStruct((NT, D_IN), _f32)])
_k4 = pl.pallas_call(
    _k4_body,
    out_shape=jax.ShapeDtypeStruct((N, D_IN), _f32))


def kernel(x, edge_index, W1l, b1, W1r, W2l, b2, W2r, W3l, b3, W3r, W4l, b4, W4r):
    src = edge_index[0]
    dst = edge_index[1]
    pad = jnp.full((EP - E,), N, jnp.int32)
    srcp = jnp.concatenate([src, pad])
    dstp = jnp.concatenate([dst, pad])
    xt = jnp.concatenate([x, jnp.zeros((NT - N, D_IN), _f32)])
    zeros2_144 = jnp.zeros((N_ACC, D_AUG), _f32)
    zeros2_128 = jnp.zeros((N_ACC, 128), _f32)
    zeros2_64 = jnp.zeros((N_ACC, 64), _f32)
    xa = jnp.concatenate(
        [xt, jnp.concatenate([jnp.ones((N, 1), _f32), jnp.zeros((NT - N, 1), _f32)]),
         jnp.zeros((NT, D_AUG - 129), _f32)], axis=1)

    e64 = _eidx(srcp, dstp, 64)
    e80 = _eidx(srcp, dstp, 80)
    e128 = _eidx(srcp, dstp, 128)

    agg1 = _agg144(xa, e64, zeros2_144)
    h, m2, inv = _k1(agg1, x, W1l, b1, W1r, W2l)
    s2 = _agg64(m2, e128, zeros2_64)
    z = _k2(s2, inv, h, W2r, b2)
    agg3 = _agg64(z, e128, zeros2_64)
    h2, m4 = _k3(agg3, inv, z, W3l, b3, W3r, W4l)
    s4 = _agg128(m4, e80, zeros2_128)
    x_hat = _k4(s4, inv, h2, W4r, b4)
    return x_hat


# speedup vs baseline: 1.3459x; 1.0631x over previous
import functools

import jax
import jax.numpy as jnp
from jax import lax
from jax.experimental import pallas as pl
from jax.experimental.pallas import tpu as pltpu
from jax.experimental.pallas import tpu_sc as plsc

N = 10000
E = 320000
D_IN = 128
D_HID = 128
D_LAT = 64

NT = N + 8
NCORE = 2
NSUB = 16
NW = NCORE * NSUB
RPT = 626
N_ACC = NSUB * RPT
EP = NW * 10240
TPB = EP // NW

_f32 = jnp.float32


def _make_agg(D, Bx, NB, split=None):
    iters = TPB // Bx
    it0 = iters if split is None else split
    it1 = 2 * iters - it0
    assert TPB % Bx == 0 and it0 % (2 * NB) == 0 and it1 % (2 * NB) == 0, (D, Bx, NB)
    pertile = NB * (Bx * D + 4 * Bx) + 64
    assert 16 * pertile + N_ACC * D < 2**21 - 16384, (D, Bx, NB)
    mesh = plsc.VectorSubcoreMesh(core_axis_name="c", subcore_axis_name="s")
    out_type = jax.ShapeDtypeStruct((NCORE, N_ACC, D), _f32)
    scratch = (
        [pltpu.VMEM((2, Bx), jnp.int32) for _ in range(2 * NB)]
        + [pltpu.VMEM((Bx, D), _f32) for _ in range(NB)]
        + [pltpu.VMEM_SHARED((N_ACC, D), _f32)]
        + [pltpu.SemaphoreType.DMA for _ in range(4 * NB)]
    )

    def body(table, eidx, zeros2, out, *rest):
        idxA = rest[:NB]
        idxB = rest[NB:2 * NB]
        rows = rest[2 * NB:3 * NB]
        acc = rest[3 * NB]
        sems = rest[3 * NB + 1:]
        isemA = sems[:NB]
        isemB = sems[NB:2 * NB]
        gsem = sems[2 * NB:3 * NB]
        ssem = sems[3 * NB:4 * NB]
        c = lax.axis_index("c")
        s = lax.axis_index("s")
        w = c * NSUB + s
        rbase = s * RPT
        g0 = jnp.where(c == 0, s * it0, NSUB * it0 + s * it1)
        nstep = jnp.where(c == 0, it0 // (2 * NB), it1 // (2 * NB))
        glast = NSUB * (it0 + it1) - 1

        pltpu.sync_copy(zeros2.at[pl.ds(rbase, RPT)], acc.at[pl.ds(rbase, RPT)])
        for b in range(NB):
            pltpu.async_copy(eidx.at[g0 + b], idxA[b], isemA[b])
        plsc.subcore_barrier()

        def half(gbase, idx, isem, prefetch_g, pidx, psem):
            for b in range(NB):
                pltpu.make_async_copy(eidx.at[gbase + b], idx[b], isem[b]).wait()
            for b in range(NB):
                pltpu.async_copy(eidx.at[lax.min(prefetch_g + b, glast)],
                                 pidx[b], psem[b])
            for b in range(NB):
                pltpu.async_copy(table.at[idx[b].at[0]], rows[b], gsem[b])
            for b in range(NB):
                pltpu.make_async_copy(table.at[idx[b].at[0]], rows[b],
                                      gsem[b]).wait()
                pltpu.async_copy(rows[b], acc.at[idx[b].at[1]], ssem[b],
                                 add=True)
            for b in range(NB):
                pltpu.make_async_copy(rows[b], acc.at[idx[b].at[1]],
                                      ssem[b]).wait()

        def step(j, carry):
            ga = g0 + 2 * j * NB
            gb = ga + NB
            half(ga, idxA, isemA, gb, idxB, isemB)
            half(gb, idxB, isemB, ga + 2 * NB, idxA, isemA)
            return carry

        lax.fori_loop(0, nstep, step, 0)
        for b in range(NB):
            pltpu.make_async_copy(eidx.at[g0], idxA[b], isemA[b]).wait()
        plsc.subcore_barrier()
        pltpu.sync_copy(acc.at[pl.ds(rbase, RPT)], out.at[c, pl.ds(rbase, RPT)])

    return pl.kernel(body, out_type=out_type, mesh=mesh, scratch_types=scratch,
                     compiler_params=pltpu.CompilerParams(use_tc_tiling_on_sc=False),
                     name=f"sc_agg_d{D}")


D_AUG = 144
_agg144 = _make_agg(D_AUG, 64, 4, split=272)
_agg128 = _make_agg(128, 80, 4, split=216)
_agg64 = _make_agg(64, 128, 4, split=136)


def _eidx(srcp, dstp, Bx):
    return jnp.stack([srcp.reshape(-1, Bx), dstp.reshape(-1, Bx)], axis=1)


def _k1_body(agg, x, w1l, b1, w1r, w2l, h_ref, m2_ref, inv_ref):
    cnt = agg[0, :N, 128] + agg[1, :N, 128]
    inv = 1.0 / jnp.maximum(cnt, 1.0)
    inv_ref[...] = inv
    a = (agg[0, :N, :128] + agg[1, :N, :128]) * inv[:, None]
    h = a @ w1l[...] + b1[...][None, :] + x[...] @ w1r[...]
    h = jnp.maximum(h, 0.0)
    h_ref[:N, :] = h
    h_ref[N:, :] = jnp.zeros((NT - N, D_HID), _f32)
    m2_ref[:N, :] = h @ w2l[...]
    m2_ref[N:, :] = jnp.zeros((NT - N, D_LAT), _f32)


def _k2_body(s2, inv, h, w2r, b2, z_ref):
    z = ((s2[0, :N, :] + s2[1, :N, :]) * inv[...][:, None]
         + b2[...][None, :] + h[:N, :] @ w2r[...])
    z_ref[:N, :] = z
    z_ref[N:, :] = jnp.zeros((NT - N, D_LAT), _f32)


def _k3_body(agg3, inv, z, w3l, b3, w3r, w4l, h2_ref, m4_ref):
    a = (agg3[0, :N, :] + agg3[1, :N, :]) * inv[...][:, None]
    h2 = a @ w3l[...] + b3[...][None, :] + z[:N, :] @ w3r[...]
    h2 = jnp.maximum(h2, 0.0)
    h2_ref[:N, :] = h2
    h2_ref[N:, :] = jnp.zeros((NT - N, D_HID), _f32)
    m4_ref[:N, :] = h2 @ w4l[...]
    m4_ref[N:, :] = jnp.zeros((NT - N, D_IN), _f32)


def _k4_body(s4, inv, h2, w4r, b4, out_ref):
    out_ref[...] = ((s4[0, :N, :] + s4[1, :N, :]) * inv[...][:, None]
                    + b4[...][None, :] + h2[:N, :] @ w4r[...])


_k1 = pl.pallas_call(
    _k1_body,
    out_shape=[jax.ShapeDtypeStruct((NT, D_HID), _f32),
               jax.ShapeDtypeStruct((NT, D_LAT), _f32),
               jax.ShapeDtypeStruct((N,), _f32)])
_k2 = pl.pallas_call(
    _k2_body,
    out_shape=jax.ShapeDtypeStruct((NT, D_LAT), _f32))
_k3 = pl.pallas_call(
    _k3_body,
    out_shape=[jax.ShapeDtypeStruct((NT, D_HID), _f32),
               jax.ShapeDtypeStruct((NT, D_IN), _f32)])
_k4 = pl.pallas_call(
    _k4_body,
    out_shape=jax.ShapeDtypeStruct((N, D_IN), _f32))


def kernel(x, edge_index, W1l, b1, W1r, W2l, b2, W2r, W3l, b3, W3r, W4l, b4, W4r):
    src = edge_index[0]
    dst = edge_index[1]
    pad = jnp.full((EP - E,), N, jnp.int32)
    srcp = jnp.concatenate([src, pad])
    dstp = jnp.concatenate([dst, pad])
    xt = jnp.concatenate([x, jnp.zeros((NT - N, D_IN), _f32)])
    zeros2_144 = jnp.zeros((N_ACC, D_AUG), _f32)
    zeros2_128 = jnp.zeros((N_ACC, 128), _f32)
    zeros2_64 = jnp.zeros((N_ACC, 64), _f32)
    xa = jnp.concatenate(
        [xt, jnp.concatenate([jnp.ones((N, 1), _f32), jnp.zeros((NT - N, 1), _f32)]),
         jnp.zeros((NT, D_AUG - 129), _f32)], axis=1)

    e64 = _eidx(srcp, dstp, 64)
    e80 = _eidx(srcp, dstp, 80)
    e128 = _eidx(srcp, dstp, 128)

    agg1 = _agg144(xa, e64, zeros2_144)
    h, m2, inv = _k1(agg1, x, W1l, b1, W1r, W2l)
    s2 = _agg64(m2, e128, zeros2_64)
    z = _k2(s2, inv, h, W2r, b2)
    agg3 = _agg64(z, e128, zeros2_64)
    h2, m4 = _k3(agg3, inv, z, W3l, b3, W3r, W4l)
    s4 = _agg128(m4, e80, zeros2_128)
    x_hat = _k4(s4, inv, h2, W4r, b4)
    return x_hat


# speedup vs baseline: 1.5369x; 1.1419x over previous
import functools

import jax
import jax.numpy as jnp
from jax import lax
from jax.experimental import pallas as pl
from jax.experimental.pallas import tpu as pltpu
from jax.experimental.pallas import tpu_sc as plsc

N = 10000
E = 320000
D_IN = 128
D_HID = 128
D_LAT = 64

NT = N + 8
NCORE = 2
NSUB = 16
NW = NCORE * NSUB
RPT = 626
N_ACC = NSUB * RPT
EP = NW * 10240
TPB = EP // NW

_f32 = jnp.float32


def _make_agg(D, Bx, NB, split=None):
    iters = TPB // Bx
    it0 = iters if split is None else split
    it1 = 2 * iters - it0
    assert TPB % Bx == 0 and it0 % (2 * NB) == 0 and it1 % (2 * NB) == 0, (D, Bx, NB)
    pertile = NB * (Bx * D + 4 * Bx) + 64
    assert 16 * pertile + N_ACC * D < 2**21 - 16384, (D, Bx, NB)
    mesh = plsc.VectorSubcoreMesh(core_axis_name="c", subcore_axis_name="s")
    out_type = jax.ShapeDtypeStruct((NCORE, N_ACC, D), _f32)
    scratch = (
        [pltpu.VMEM((2, Bx), jnp.int32) for _ in range(2 * NB)]
        + [pltpu.VMEM((Bx, D), _f32) for _ in range(NB)]
        + [pltpu.VMEM_SHARED((N_ACC, D), _f32)]
        + [pltpu.SemaphoreType.DMA for _ in range(4 * NB)]
    )

    def body(table, eidx, zeros2, out, *rest):
        idxA = rest[:NB]
        idxB = rest[NB:2 * NB]
        rows = rest[2 * NB:3 * NB]
        acc = rest[3 * NB]
        sems = rest[3 * NB + 1:]
        isemA = sems[:NB]
        isemB = sems[NB:2 * NB]
        gsem = sems[2 * NB:3 * NB]
        ssem = sems[3 * NB:4 * NB]
        c = lax.axis_index("c")
        s = lax.axis_index("s")
        w = c * NSUB + s
        rbase = s * RPT
        g0 = jnp.where(c == 0, s * it0, NSUB * it0 + s * it1)
        nstep = jnp.where(c == 0, it0 // (2 * NB), it1 // (2 * NB))
        glast = NSUB * (it0 + it1) - 1

        pltpu.sync_copy(zeros2.at[pl.ds(rbase, RPT)], acc.at[pl.ds(rbase, RPT)])
        for b in range(NB):
            pltpu.async_copy(eidx.at[g0 + b], idxA[b], isemA[b])
        plsc.subcore_barrier()

        def half(gbase, idx, isem, prefetch_g, pidx, psem):
            for b in range(NB):
                pltpu.make_async_copy(eidx.at[gbase + b], idx[b], isem[b]).wait()
            for b in range(NB):
                pltpu.async_copy(eidx.at[lax.min(prefetch_g + b, glast)],
                                 pidx[b], psem[b])
            for b in range(NB):
                pltpu.async_copy(table.at[idx[b].at[0]], rows[b], gsem[b])
            for b in range(NB):
                pltpu.make_async_copy(table.at[idx[b].at[0]], rows[b],
                                      gsem[b]).wait()
                pltpu.async_copy(rows[b], acc.at[idx[b].at[1]], ssem[b],
                                 add=True)
            for b in range(NB):
                pltpu.make_async_copy(rows[b], acc.at[idx[b].at[1]],
                                      ssem[b]).wait()

        def step(j, carry):
            ga = g0 + 2 * j * NB
            gb = ga + NB
            half(ga, idxA, isemA, gb, idxB, isemB)
            half(gb, idxB, isemB, ga + 2 * NB, idxA, isemA)
            return carry

        lax.fori_loop(0, nstep, step, 0)
        for b in range(NB):
            pltpu.make_async_copy(eidx.at[g0], idxA[b], isemA[b]).wait()
        plsc.subcore_barrier()
        pltpu.sync_copy(acc.at[pl.ds(rbase, RPT)], out.at[c, pl.ds(rbase, RPT)])

    return pl.kernel(body, out_type=out_type, mesh=mesh, scratch_types=scratch,
                     compiler_params=pltpu.CompilerParams(use_tc_tiling_on_sc=False),
                     name=f"sc_agg_d{D}")


D_AUG = 144
_agg144 = _make_agg(D_AUG, 64, 4, split=288)
_agg128 = _make_agg(128, 80, 4, split=232)
_agg64 = _make_agg(64, 128, 4, split=144)


def _eidx(srcp, dstp, Bx):
    return jnp.stack([srcp.reshape(-1, Bx), dstp.reshape(-1, Bx)], axis=1)


def _k1_body(agg, x, w1l, b1, w1r, w2l, h_ref, m2_ref, inv_ref):
    cnt = agg[0, :N, 128] + agg[1, :N, 128]
    inv = 1.0 / jnp.maximum(cnt, 1.0)
    inv_ref[...] = inv
    a = (agg[0, :N, :128] + agg[1, :N, :128]) * inv[:, None]
    h = a @ w1l[...] + b1[...][None, :] + x[...] @ w1r[...]
    h = jnp.maximum(h, 0.0)
    h_ref[:N, :] = h
    h_ref[N:, :] = jnp.zeros((NT - N, D_HID), _f32)
    m2_ref[:N, :] = h @ w2l[...]
    m2_ref[N:, :] = jnp.zeros((NT - N, D_LAT), _f32)


def _k2_body(s2, inv, h, w2r, b2, z_ref):
    z = ((s2[0, :N, :] + s2[1, :N, :]) * inv[...][:, None]
         + b2[...][None, :] + h[:N, :] @ w2r[...])
    z_ref[:N, :] = z
    z_ref[N:, :] = jnp.zeros((NT - N, D_LAT), _f32)


def _k3_body(agg3, inv, z, w3l, b3, w3r, w4l, h2_ref, m4_ref):
    a = (agg3[0, :N, :] + agg3[1, :N, :]) * inv[...][:, None]
    h2 = a @ w3l[...] + b3[...][None, :] + z[:N, :] @ w3r[...]
    h2 = jnp.maximum(h2, 0.0)
    h2_ref[:N, :] = h2
    h2_ref[N:, :] = jnp.zeros((NT - N, D_HID), _f32)
    m4_ref[:N, :] = h2 @ w4l[...]
    m4_ref[N:, :] = jnp.zeros((NT - N, D_IN), _f32)


def _k4_body(s4, inv, h2, w4r, b4, out_ref):
    out_ref[...] = ((s4[0, :N, :] + s4[1, :N, :]) * inv[...][:, None]
                    + b4[...][None, :] + h2[:N, :] @ w4r[...])


_k1 = pl.pallas_call(
    _k1_body,
    out_shape=[jax.ShapeDtypeStruct((NT, D_HID), _f32),
               jax.ShapeDtypeStruct((NT, D_LAT), _f32),
               jax.ShapeDtypeStruct((N,), _f32)])
_k2 = pl.pallas_call(
    _k2_body,
    out_shape=jax.ShapeDtypeStruct((NT, D_LAT), _f32))
_k3 = pl.pallas_call(
    _k3_body,
    out_shape=[jax.ShapeDtypeStruct((NT, D_HID), _f32),
               jax.ShapeDtypeStruct((NT, D_IN), _f32)])
_k4 = pl.pallas_call(
    _k4_body,
    out_shape=jax.ShapeDtypeStruct((N, D_IN), _f32))


def kernel(x, edge_index, W1l, b1, W1r, W2l, b2, W2r, W3l, b3, W3r, W4l, b4, W4r):
    src = edge_index[0]
    dst = edge_index[1]
    pad = jnp.full((EP - E,), N, jnp.int32)
    srcp = jnp.concatenate([src, pad])
    dstp = jnp.concatenate([dst, pad])
    xt = jnp.concatenate([x, jnp.zeros((NT - N, D_IN), _f32)])
    zeros2_144 = jnp.zeros((N_ACC, D_AUG), _f32)
    zeros2_128 = jnp.zeros((N_ACC, 128), _f32)
    zeros2_64 = jnp.zeros((N_ACC, 64), _f32)
    xa = jnp.concatenate(
        [xt, jnp.concatenate([jnp.ones((N, 1), _f32), jnp.zeros((NT - N, 1), _f32)]),
         jnp.zeros((NT, D_AUG - 129), _f32)], axis=1)

    e64 = _eidx(srcp, dstp, 64)
    e80 = _eidx(srcp, dstp, 80)
    e128 = _eidx(srcp, dstp, 128)

    agg1 = _agg144(xa, e64, zeros2_144)
    h, m2, inv = _k1(agg1, x, W1l, b1, W1r, W2l)
    s2 = _agg64(m2, e128, zeros2_64)
    z = _k2(s2, inv, h, W2r, b2)
    agg3 = _agg64(z, e128, zeros2_64)
    h2, m4 = _k3(agg3, inv, z, W3l, b3, W3r, W4l)
    s4 = _agg128(m4, e80, zeros2_128)
    x_hat = _k4(s4, inv, h2, W4r, b4)
    return x_hat


# speedup vs baseline: 1.5526x; 1.0102x over previous
import functools

import jax
import jax.numpy as jnp
from jax import lax
from jax.experimental import pallas as pl
from jax.experimental.pallas import tpu as pltpu
from jax.experimental.pallas import tpu_sc as plsc

N = 10000
E = 320000
D_IN = 128
D_HID = 128
D_LAT = 64

NT = N + 8
NCORE = 2
NSUB = 16
NW = NCORE * NSUB
RPT = 626
N_ACC = NSUB * RPT
EP = NW * 10240
TPB = EP // NW

_f32 = jnp.float32


def _make_agg(D, Bx, NB, split=None):
    iters = TPB // Bx
    it0 = iters if split is None else split
    it1 = 2 * iters - it0
    assert TPB % Bx == 0 and it0 % (2 * NB) == 0 and it1 % (2 * NB) == 0, (D, Bx, NB)
    pertile = NB * (Bx * D + 4 * Bx) + 64
    assert 16 * pertile + N_ACC * D < 2**21 - 16384, (D, Bx, NB)
    mesh = plsc.VectorSubcoreMesh(core_axis_name="c", subcore_axis_name="s")
    out_type = jax.ShapeDtypeStruct((NCORE, N_ACC, D), _f32)
    scratch = (
        [pltpu.VMEM((2, Bx), jnp.int32) for _ in range(2 * NB)]
        + [pltpu.VMEM((Bx, D), _f32) for _ in range(NB)]
        + [pltpu.VMEM_SHARED((N_ACC, D), _f32)]
        + [pltpu.SemaphoreType.DMA for _ in range(4 * NB)]
    )

    def body(table, eidx, zeros2, out, *rest):
        idxA = rest[:NB]
        idxB = rest[NB:2 * NB]
        rows = rest[2 * NB:3 * NB]
        acc = rest[3 * NB]
        sems = rest[3 * NB + 1:]
        isemA = sems[:NB]
        isemB = sems[NB:2 * NB]
        gsem = sems[2 * NB:3 * NB]
        ssem = sems[3 * NB:4 * NB]
        c = lax.axis_index("c")
        s = lax.axis_index("s")
        w = c * NSUB + s
        rbase = s * RPT
        g0 = jnp.where(c == 0, s * it0, NSUB * it0 + s * it1)
        nstep = jnp.where(c == 0, it0 // (2 * NB), it1 // (2 * NB))
        glast = NSUB * (it0 + it1) - 1

        pltpu.sync_copy(zeros2.at[pl.ds(rbase, RPT)], acc.at[pl.ds(rbase, RPT)])
        for b in range(NB):
            pltpu.async_copy(eidx.at[g0 + b], idxA[b], isemA[b])
        plsc.subcore_barrier()

        def half(gbase, idx, isem, prefetch_g, pidx, psem):
            for b in range(NB):
                pltpu.make_async_copy(eidx.at[gbase + b], idx[b], isem[b]).wait()
            for b in range(NB):
                pltpu.async_copy(eidx.at[lax.min(prefetch_g + b, glast)],
                                 pidx[b], psem[b])
            for b in range(NB):
                pltpu.async_copy(table.at[idx[b].at[0]], rows[b], gsem[b])
            for b in range(NB):
                pltpu.make_async_copy(table.at[idx[b].at[0]], rows[b],
                                      gsem[b]).wait()
                pltpu.async_copy(rows[b], acc.at[idx[b].at[1]], ssem[b],
                                 add=True)
            for b in range(NB):
                pltpu.make_async_copy(rows[b], acc.at[idx[b].at[1]],
                                      ssem[b]).wait()

        def step(j, carry):
            ga = g0 + 2 * j * NB
            gb = ga + NB
            half(ga, idxA, isemA, gb, idxB, isemB)
            half(gb, idxB, isemB, ga + 2 * NB, idxA, isemA)
            return carry

        lax.fori_loop(0, nstep, step, 0)
        for b in range(NB):
            pltpu.make_async_copy(eidx.at[g0], idxA[b], isemA[b]).wait()
        plsc.subcore_barrier()
        pltpu.sync_copy(acc.at[pl.ds(rbase, RPT)], out.at[c, pl.ds(rbase, RPT)])

    return pl.kernel(body, out_type=out_type, mesh=mesh, scratch_types=scratch,
                     compiler_params=pltpu.CompilerParams(use_tc_tiling_on_sc=False),
                     name=f"sc_agg_d{D}")


D_AUG = 144
_agg144 = _make_agg(D_AUG, 64, 4, split=304)
_agg128 = _make_agg(128, 80, 4, split=248)
_agg64 = _make_agg(64, 128, 4, split=152)


def _eidx(srcp, dstp, Bx):
    return jnp.stack([srcp.reshape(-1, Bx), dstp.reshape(-1, Bx)], axis=1)


def _k1_body(agg, x, w1l, b1, w1r, w2l, h_ref, m2_ref, inv_ref):
    cnt = agg[0, :N, 128] + agg[1, :N, 128]
    inv = 1.0 / jnp.maximum(cnt, 1.0)
    inv_ref[...] = inv
    a = (agg[0, :N, :128] + agg[1, :N, :128]) * inv[:, None]
    h = a @ w1l[...] + b1[...][None, :] + x[...] @ w1r[...]
    h = jnp.maximum(h, 0.0)
    h_ref[:N, :] = h
    h_ref[N:, :] = jnp.zeros((NT - N, D_HID), _f32)
    m2_ref[:N, :] = h @ w2l[...]
    m2_ref[N:, :] = jnp.zeros((NT - N, D_LAT), _f32)


def _k2_body(s2, inv, h, w2r, b2, z_ref):
    z = ((s2[0, :N, :] + s2[1, :N, :]) * inv[...][:, None]
         + b2[...][None, :] + h[:N, :] @ w2r[...])
    z_ref[:N, :] = z
    z_ref[N:, :] = jnp.zeros((NT - N, D_LAT), _f32)


def _k3_body(agg3, inv, z, w3l, b3, w3r, w4l, h2_ref, m4_ref):
    a = (agg3[0, :N, :] + agg3[1, :N, :]) * inv[...][:, None]
    h2 = a @ w3l[...] + b3[...][None, :] + z[:N, :] @ w3r[...]
    h2 = jnp.maximum(h2, 0.0)
    h2_ref[:N, :] = h2
    h2_ref[N:, :] = jnp.zeros((NT - N, D_HID), _f32)
    m4_ref[:N, :] = h2 @ w4l[...]
    m4_ref[N:, :] = jnp.zeros((NT - N, D_IN), _f32)


def _k4_body(s4, inv, h2, w4r, b4, out_ref):
    out_ref[...] = ((s4[0, :N, :] + s4[1, :N, :]) * inv[...][:, None]
                    + b4[...][None, :] + h2[:N, :] @ w4r[...])


_k1 = pl.pallas_call(
    _k1_body,
    out_shape=[jax.ShapeDtypeStruct((NT, D_HID), _f32),
               jax.ShapeDtypeStruct((NT, D_LAT), _f32),
               jax.ShapeDtypeStruct((N,), _f32)])
_k2 = pl.pallas_call(
    _k2_body,
    out_shape=jax.ShapeDtypeStruct((NT, D_LAT), _f32))
_k3 = pl.pallas_call(
    _k3_body,
    out_shape=[jax.ShapeDtypeStruct((NT, D_HID), _f32),
               jax.ShapeDtypeStruct((NT, D_IN), _f32)])
_k4 = pl.pallas_call(
    _k4_body,
    out_shape=jax.ShapeDtypeStruct((N, D_IN), _f32))


def kernel(x, edge_index, W1l, b1, W1r, W2l, b2, W2r, W3l, b3, W3r, W4l, b4, W4r):
    src = edge_index[0]
    dst = edge_index[1]
    pad = jnp.full((EP - E,), N, jnp.int32)
    srcp = jnp.concatenate([src, pad])
    dstp = jnp.concatenate([dst, pad])
    xt = jnp.concatenate([x, jnp.zeros((NT - N, D_IN), _f32)])
    zeros2_144 = jnp.zeros((N_ACC, D_AUG), _f32)
    zeros2_128 = jnp.zeros((N_ACC, 128), _f32)
    zeros2_64 = jnp.zeros((N_ACC, 64), _f32)
    xa = jnp.concatenate(
        [xt, jnp.concatenate([jnp.ones((N, 1), _f32), jnp.zeros((NT - N, 1), _f32)]),
         jnp.zeros((NT, D_AUG - 129), _f32)], axis=1)

    e64 = _eidx(srcp, dstp, 64)
    e80 = _eidx(srcp, dstp, 80)
    e128 = _eidx(srcp, dstp, 128)

    agg1 = _agg144(xa, e64, zeros2_144)
    h, m2, inv = _k1(agg1, x, W1l, b1, W1r, W2l)
    s2 = _agg64(m2, e128, zeros2_64)
    z = _k2(s2, inv, h, W2r, b2)
    agg3 = _agg64(z, e128, zeros2_64)
    h2, m4 = _k3(agg3, inv, z, W3l, b3, W3r, W4l)
    s4 = _agg128(m4, e80, zeros2_128)
    x_hat = _k4(s4, inv, h2, W4r, b4)
    return x_hat
